# Initial kernel scaffold; baseline (speedup 1.0000x reference)
import jax
import jax.numpy as jnp
from jax.experimental import pallas as pl


def kernel(clap_embeddings, codebooks):
    raise NotImplementedError("write your pallas kernel here")


# speedup vs baseline: 2.0579x; 2.0579x over previous
import jax
import jax.numpy as jnp
from jax.experimental import pallas as pl
from jax.experimental.pallas import tpu as pltpu

_D = 512
_K = 1024
_Q = 8
_TILE = 512


def _rvq_body(x_ref, c1_ref, c2_ref, c3_ref, cnorm_ref,
              qout_ref, idx_ref, loss_ref):
    residual = x_ref[0]
    quant_acc = jnp.zeros_like(residual)
    iota = jax.lax.broadcasted_iota(jnp.int32, (_TILE, _K), 1)
    for q in range(_Q):
        fnorm = jnp.sum(residual * residual, axis=1, keepdims=True)
        mm = jax.lax.dot_general(
            residual.astype(jnp.bfloat16), c1_ref[q], (((1,), (1,)), ((), ())),
            preferred_element_type=jnp.float32)
        d2 = (fnorm - 2.0 * mm) + cnorm_ref[q]
        minval = jnp.min(d2, axis=1, keepdims=True)
        idxk = jnp.min(jnp.where(d2 == minval, iota, _K),
                       axis=1, keepdims=True)
        onehot = (iota == idxk).astype(jnp.bfloat16)
        q1 = jax.lax.dot_general(onehot, c1_ref[q], (((1,), (0,)), ((), ())),
                                 preferred_element_type=jnp.float32)
        q2 = jax.lax.dot_general(onehot, c2_ref[q], (((1,), (0,)), ((), ())),
                                 preferred_element_type=jnp.float32)
        q3 = jax.lax.dot_general(onehot, c3_ref[q], (((1,), (0,)), ((), ())),
                                 preferred_element_type=jnp.float32)
        quant = (q1 + q2) + q3
        rmq = residual - quant
        loss_ref[0, q:q + 1, :] = jnp.sum(rmq * rmq, keepdims=True)
        quant_acc = quant_acc + (residual + (quant - residual))
        residual = rmq
        idx_ref[0, :, q:q + 1] = idxk
    qout_ref[0] = quant_acc


def kernel(clap_embeddings, codebooks):
    x = clap_embeddings
    B, T, D = x.shape
    n_tiles = (B * T) // _TILE

    c1 = codebooks.astype(jnp.bfloat16)
    r1 = codebooks - jax.lax.optimization_barrier(c1).astype(jnp.float32)
    c2 = r1.astype(jnp.bfloat16)
    r2 = r1 - jax.lax.optimization_barrier(c2).astype(jnp.float32)
    c3 = r2.astype(jnp.bfloat16)
    cnorm = jnp.sum(codebooks * codebooks, axis=2)[:, None, :]

    xt = x.reshape(n_tiles, _TILE, D)

    const3 = lambda i: (0, 0, 0)
    qout, idx, loss = pl.pallas_call(
        _rvq_body,
        grid=(n_tiles,),
        in_specs=[
            pl.BlockSpec((1, _TILE, _D), lambda i: (i, 0, 0)),
            pl.BlockSpec((_Q, _K, _D), const3),
            pl.BlockSpec((_Q, _K, _D), const3),
            pl.BlockSpec((_Q, _K, _D), const3),
            pl.BlockSpec((_Q, 1, _K), const3),
        ],
        out_specs=[
            pl.BlockSpec((1, _TILE, _D), lambda i: (i, 0, 0)),
            pl.BlockSpec((1, _TILE, _Q), lambda i: (i, 0, 0)),
            pl.BlockSpec((1, _Q, 1), lambda i: (i, 0, 0)),
        ],
        out_shape=[
            jax.ShapeDtypeStruct((n_tiles, _TILE, _D), jnp.float32),
            jax.ShapeDtypeStruct((n_tiles, _TILE, _Q), jnp.int32),
            jax.ShapeDtypeStruct((n_tiles, _Q, 1), jnp.float32),
        ],
        compiler_params=pltpu.CompilerParams(
            dimension_semantics=("arbitrary",),
            vmem_limit_bytes=60000 * 1024,
        ),
    )(xt, c1, c2, c3, cnorm)

    quantized_out = qout.reshape(B, T, D)
    indices = idx.reshape(B, T, _Q)
    commit_loss = jnp.sum(loss) / jnp.float32(B * T * D)
    return quantized_out, indices, commit_loss


# speedup vs baseline: 2.1111x; 1.0258x over previous
import jax
import jax.numpy as jnp
from jax.experimental import pallas as pl
from jax.experimental.pallas import tpu as pltpu

_D = 512
_K = 1024
_Q = 8
_TILE = 512


_H = _TILE // 2


def _rvq_body(x_ref, c1_ref, c2_ref, c3_ref, cnorm_ref,
              qout_ref, idx_ref, loss_ref):
    iota = jax.lax.broadcasted_iota(jnp.int32, (_H, _K), 1)
    residual = [x_ref[0, :_H], x_ref[0, _H:]]
    quant_acc = [jnp.zeros_like(residual[0]) for _ in range(2)]
    fnorm = [jnp.sum(r * r, axis=1, keepdims=True) for r in residual]
    idx_cols = [[], []]
    loss_rows = []
    for q in range(_Q):
        mm = [jax.lax.dot_general(
            residual[h].astype(jnp.bfloat16), c1_ref[q],
            (((1,), (1,)), ((), ())),
            preferred_element_type=jnp.float32) for h in range(2)]
        loss_q = []
        for h in range(2):
            d2 = (fnorm[h] - 2.0 * mm[h]) + cnorm_ref[q]
            minval = jnp.min(d2, axis=1, keepdims=True)
            idxk = jnp.min(jnp.where(d2 == minval, iota, _K),
                           axis=1, keepdims=True)
            onehot = (iota == idxk).astype(jnp.bfloat16)
            q1 = jax.lax.dot_general(onehot, c1_ref[q],
                                     (((1,), (0,)), ((), ())),
                                     preferred_element_type=jnp.float32)
            q2 = jax.lax.dot_general(onehot, c2_ref[q],
                                     (((1,), (0,)), ((), ())),
                                     preferred_element_type=jnp.float32)
            q3 = jax.lax.dot_general(onehot, c3_ref[q],
                                     (((1,), (0,)), ((), ())),
                                     preferred_element_type=jnp.float32)
            quant = (q1 + q2) + q3
            rmq = residual[h] - quant
            rn = jnp.sum(rmq * rmq, axis=1, keepdims=True)
            loss_q.append(jnp.sum(rn, keepdims=True))
            quant_acc[h] = quant_acc[h] + (residual[h] + (quant - residual[h]))
            residual[h] = rmq
            fnorm[h] = rn
            idx_cols[h].append(idxk)
        loss_rows.append(loss_q[0] + loss_q[1])
    qout_ref[0] = jnp.concatenate([quant_acc[0], quant_acc[1]], axis=0)
    idx_ref[0] = jnp.concatenate(
        [jnp.concatenate(idx_cols[h], axis=1) for h in range(2)], axis=0)
    loss_ref[0] = jnp.concatenate(loss_rows, axis=0)


def kernel(clap_embeddings, codebooks):
    x = clap_embeddings
    B, T, D = x.shape
    n_tiles = (B * T) // _TILE

    c1 = codebooks.astype(jnp.bfloat16)
    r1 = codebooks - jax.lax.optimization_barrier(c1).astype(jnp.float32)
    c2 = r1.astype(jnp.bfloat16)
    r2 = r1 - jax.lax.optimization_barrier(c2).astype(jnp.float32)
    c3 = r2.astype(jnp.bfloat16)
    cnorm = jnp.sum(codebooks * codebooks, axis=2)[:, None, :]

    xt = x.reshape(n_tiles, _TILE, D)

    const3 = lambda i: (0, 0, 0)
    qout, idx, loss = pl.pallas_call(
        _rvq_body,
        grid=(n_tiles,),
        in_specs=[
            pl.BlockSpec((1, _TILE, _D), lambda i: (i, 0, 0)),
            pl.BlockSpec((_Q, _K, _D), const3),
            pl.BlockSpec((_Q, _K, _D), const3),
            pl.BlockSpec((_Q, _K, _D), const3),
            pl.BlockSpec((_Q, 1, _K), const3),
        ],
        out_specs=[
            pl.BlockSpec((1, _TILE, _D), lambda i: (i, 0, 0)),
            pl.BlockSpec((1, _TILE, _Q), lambda i: (i, 0, 0)),
            pl.BlockSpec((1, _Q, 1), lambda i: (i, 0, 0)),
        ],
        out_shape=[
            jax.ShapeDtypeStruct((n_tiles, _TILE, _D), jnp.float32),
            jax.ShapeDtypeStruct((n_tiles, _TILE, _Q), jnp.int32),
            jax.ShapeDtypeStruct((n_tiles, _Q, 1), jnp.float32),
        ],
        compiler_params=pltpu.CompilerParams(
            dimension_semantics=("arbitrary",),
            vmem_limit_bytes=60000 * 1024,
        ),
    )(xt, c1, c2, c3, cnorm)

    quantized_out = qout.reshape(B, T, D)
    indices = idx.reshape(B, T, _Q)
    commit_loss = jnp.sum(loss) / jnp.float32(B * T * D)
    return quantized_out, indices, commit_loss
